# baseline (speedup 1.0000x reference)
import jax
import jax.numpy as jnp
from jax import lax
from jax.experimental import pallas as pl


def _fused_kernel(split, split_dim, x_ref, w1_ref, b1_ref, w2_ref, b2_ref,
                  w3_ref, b3_ref, ct_ref, z_ref, idx_ref):
    x = x_ref[...]
    h = jnp.dot(x, w1_ref[...], preferred_element_type=jnp.float32) + b1_ref[...]
    h = jnp.where(h >= 0, h, 0.2 * h)
    h = jnp.dot(h, w2_ref[...], preferred_element_type=jnp.float32) + b2_ref[...]
    h = jnp.where(h >= 0, h, 0.2 * h)
    z = jnp.dot(h, w3_ref[...], preferred_element_type=jnp.float32) + b3_ref[...]
    z_ref[...] = z

    ct = ct_ref[...]
    k = ct.shape[1]
    bn = z.shape[0]
    code_sq = jnp.sum(ct * ct, axis=0, keepdims=True)
    ct_m2 = -2.0 * ct

    nc = k // 128
    iota_l = lax.broadcasted_iota(jnp.int32, (bn, 128), 1).astype(jnp.float32)
    idx_rows = []
    for j in range(split):
        v = z[:, j * split_dim:(j + 1) * split_dim]
        s = jnp.dot(v, ct_m2, preferred_element_type=jnp.float32) + code_sq
        m1 = s[:, 0:128]
        for c in range(1, nc):
            m1 = jnp.minimum(m1, s[:, c * 128:(c + 1) * 128])
        c1 = jnp.full((bn, 128), float(nc), dtype=jnp.float32)
        for c in range(nc):
            hit = s[:, c * 128:(c + 1) * 128] == m1
            c1 = jnp.minimum(c1, jnp.where(hit, float(c), float(nc)))
        m = jnp.min(m1, axis=1, keepdims=True)
        k_l = c1 * 128.0 + iota_l
        idx_f = jnp.min(jnp.where(m1 == m, k_l, float(2 * k)), axis=1)
        idx_rows.append(idx_f)
    idx_ref[...] = jnp.stack(idx_rows, axis=0).astype(jnp.int32)


def kernel(x, W1, b1, W2, b2, W3, b3, codebook):
    n, input_dim = x.shape
    d1 = W1.shape[1]
    d2 = W2.shape[1]
    z_dim = W3.shape[1]
    k, split_dim = codebook.shape
    split = z_dim // split_dim

    bn = 512
    n_blocks = n // bn

    ct = codebook.T
    b1r = b1.reshape(1, d1)
    b2r = b2.reshape(1, d2)
    b3r = b3.reshape(1, z_dim)

    import functools
    body = functools.partial(_fused_kernel, split, split_dim)
    z, idxs = pl.pallas_call(
        body,
        grid=(n_blocks,),
        in_specs=[
            pl.BlockSpec((bn, input_dim), lambda i: (i, 0)),
            pl.BlockSpec((input_dim, d1), lambda i: (0, 0)),
            pl.BlockSpec((1, d1), lambda i: (0, 0)),
            pl.BlockSpec((d1, d2), lambda i: (0, 0)),
            pl.BlockSpec((1, d2), lambda i: (0, 0)),
            pl.BlockSpec((d2, z_dim), lambda i: (0, 0)),
            pl.BlockSpec((1, z_dim), lambda i: (0, 0)),
            pl.BlockSpec((split_dim, k), lambda i: (0, 0)),
        ],
        out_specs=[
            pl.BlockSpec((bn, z_dim), lambda i: (i, 0)),
            pl.BlockSpec((split, bn), lambda i: (0, i)),
        ],
        out_shape=[
            jax.ShapeDtypeStruct((n, z_dim), jnp.float32),
            jax.ShapeDtypeStruct((split, n), jnp.int32),
        ],
    )(x, W1, b1r, W2, b2r, W3, b3r, ct)

    indices = idxs.T.astype(jnp.int64)
    return (z, indices)


# speedup vs baseline: 1.5285x; 1.5285x over previous
import jax
import jax.numpy as jnp
from jax import lax
from jax.experimental import pallas as pl


def _fused_kernel(split, split_dim, x_ref, w1_ref, b1_ref, w2_ref, b2_ref,
                  w3_ref, b3_ref, ct_ref, z_ref, idx_ref):
    x = x_ref[...]
    h = jnp.dot(x, w1_ref[...], preferred_element_type=jnp.float32) + b1_ref[...]
    h = jnp.where(h >= 0, h, 0.2 * h)
    h = jnp.dot(h, w2_ref[...], preferred_element_type=jnp.float32) + b2_ref[...]
    h = jnp.where(h >= 0, h, 0.2 * h)
    z = jnp.dot(h, w3_ref[...], preferred_element_type=jnp.float32) + b3_ref[...]
    z_ref[...] = z

    ct = ct_ref[...]
    k = ct.shape[1]
    bn = z.shape[0]
    code_sq = jnp.sum(ct * ct, axis=0, keepdims=True)
    ct_m2 = -2.0 * ct

    nc = k // 128
    br = 64
    iota_l = lax.broadcasted_iota(jnp.int32, (bn, 128), 1).astype(jnp.float32)
    idx_rows = []
    for j in range(split):
        v = z[:, j * split_dim:(j + 1) * split_dim]
        raw = jnp.dot(v, ct_m2, preferred_element_type=jnp.float32)
        m1_blocks, c1_blocks = [], []
        for rb in range(0, bn, br):
            run_v = raw[rb:rb + br, 0:128] + code_sq[:, 0:128]
            run_c = jnp.zeros((br, 128), dtype=jnp.float32)
            for c in range(1, nc):
                t = raw[rb:rb + br, c * 128:(c + 1) * 128] \
                    + code_sq[:, c * 128:(c + 1) * 128]
                lt = t < run_v
                run_v = jnp.where(lt, t, run_v)
                run_c = jnp.where(lt, float(c), run_c)
            m1_blocks.append(run_v)
            c1_blocks.append(run_c)
        m1 = jnp.concatenate(m1_blocks, axis=0)
        c1 = jnp.concatenate(c1_blocks, axis=0)
        m = jnp.min(m1, axis=1, keepdims=True)
        k_l = c1 * 128.0 + iota_l
        idx_f = jnp.min(jnp.where(m1 == m, k_l, float(2 * k)), axis=1)
        idx_rows.append(idx_f)
    idx_ref[...] = jnp.stack(idx_rows, axis=0).astype(jnp.int32)


def kernel(x, W1, b1, W2, b2, W3, b3, codebook):
    n, input_dim = x.shape
    d1 = W1.shape[1]
    d2 = W2.shape[1]
    z_dim = W3.shape[1]
    k, split_dim = codebook.shape
    split = z_dim // split_dim

    bn = 512
    n_blocks = n // bn

    ct = codebook.T
    b1r = b1.reshape(1, d1)
    b2r = b2.reshape(1, d2)
    b3r = b3.reshape(1, z_dim)

    import functools
    body = functools.partial(_fused_kernel, split, split_dim)
    z, idxs = pl.pallas_call(
        body,
        grid=(n_blocks,),
        in_specs=[
            pl.BlockSpec((bn, input_dim), lambda i: (i, 0)),
            pl.BlockSpec((input_dim, d1), lambda i: (0, 0)),
            pl.BlockSpec((1, d1), lambda i: (0, 0)),
            pl.BlockSpec((d1, d2), lambda i: (0, 0)),
            pl.BlockSpec((1, d2), lambda i: (0, 0)),
            pl.BlockSpec((d2, z_dim), lambda i: (0, 0)),
            pl.BlockSpec((1, z_dim), lambda i: (0, 0)),
            pl.BlockSpec((split_dim, k), lambda i: (0, 0)),
        ],
        out_specs=[
            pl.BlockSpec((bn, z_dim), lambda i: (i, 0)),
            pl.BlockSpec((split, bn), lambda i: (0, i)),
        ],
        out_shape=[
            jax.ShapeDtypeStruct((n, z_dim), jnp.float32),
            jax.ShapeDtypeStruct((split, n), jnp.int32),
        ],
    )(x, W1, b1r, W2, b2r, W3, b3r, ct)

    indices = idxs.T.astype(jnp.int64)
    return (z, indices)


# speedup vs baseline: 1.8106x; 1.1846x over previous
import jax
import jax.numpy as jnp
from jax import lax
from jax.experimental import pallas as pl


def _fused_kernel(split, split_dim, x_ref, w1_ref, b1_ref, w2_ref, b2_ref,
                  w3_ref, b3_ref, ct_ref, z_ref, idx_ref):
    x = x_ref[...]
    h = jnp.dot(x, w1_ref[...], preferred_element_type=jnp.float32) + b1_ref[...]
    h = jnp.where(h >= 0, h, 0.2 * h)
    h = jnp.dot(h, w2_ref[...], preferred_element_type=jnp.float32) + b2_ref[...]
    h = jnp.where(h >= 0, h, 0.2 * h)
    z = jnp.dot(h, w3_ref[...], preferred_element_type=jnp.float32) + b3_ref[...]
    z_ref[...] = z

    ct = ct_ref[...]
    k = ct.shape[1]
    bn = z.shape[0]
    code_sq = jnp.sum(ct * ct, axis=0, keepdims=True)
    csq_hi = code_sq.astype(jnp.bfloat16).astype(jnp.float32)
    csq_lo = code_sq - csq_hi
    pad = (-(split_dim + 2)) % 8
    ct_aug = jnp.concatenate(
        [-2.0 * ct, csq_hi, csq_lo,
         jnp.zeros((pad, k), dtype=jnp.float32)], axis=0)
    aug_cols = jnp.concatenate(
        [jnp.ones((bn, 2), dtype=jnp.float32),
         jnp.zeros((bn, pad), dtype=jnp.float32)], axis=1)

    nc = k // 128
    br = 64
    iota_l = lax.broadcasted_iota(jnp.int32, (bn, 128), 1).astype(jnp.float32)
    idx_rows = []
    for j in range(split):
        v = z[:, j * split_dim:(j + 1) * split_dim]
        va = jnp.concatenate([v, aug_cols], axis=1)
        raw = jnp.dot(va, ct_aug, preferred_element_type=jnp.float32)
        m1_blocks, c1_blocks = [], []
        for rb in range(0, bn, br):
            run_v = raw[rb:rb + br, 0:128]
            run_c = jnp.zeros((br, 128), dtype=jnp.float32)
            for c in range(1, nc):
                t = raw[rb:rb + br, c * 128:(c + 1) * 128]
                lt = t < run_v
                run_v = jnp.where(lt, t, run_v)
                run_c = jnp.where(lt, float(c), run_c)
            m1_blocks.append(run_v)
            c1_blocks.append(run_c)
        m1 = jnp.concatenate(m1_blocks, axis=0)
        c1 = jnp.concatenate(c1_blocks, axis=0)
        m = jnp.min(m1, axis=1, keepdims=True)
        k_l = c1 * 128.0 + iota_l
        idx_f = jnp.min(jnp.where(m1 == m, k_l, float(2 * k)), axis=1)
        idx_rows.append(idx_f)
    idx_ref[...] = jnp.stack(idx_rows, axis=0).astype(jnp.int32)


def kernel(x, W1, b1, W2, b2, W3, b3, codebook):
    n, input_dim = x.shape
    d1 = W1.shape[1]
    d2 = W2.shape[1]
    z_dim = W3.shape[1]
    k, split_dim = codebook.shape
    split = z_dim // split_dim

    bn = 512
    n_blocks = n // bn

    ct = codebook.T
    b1r = b1.reshape(1, d1)
    b2r = b2.reshape(1, d2)
    b3r = b3.reshape(1, z_dim)

    import functools
    body = functools.partial(_fused_kernel, split, split_dim)
    z, idxs = pl.pallas_call(
        body,
        grid=(n_blocks,),
        in_specs=[
            pl.BlockSpec((bn, input_dim), lambda i: (i, 0)),
            pl.BlockSpec((input_dim, d1), lambda i: (0, 0)),
            pl.BlockSpec((1, d1), lambda i: (0, 0)),
            pl.BlockSpec((d1, d2), lambda i: (0, 0)),
            pl.BlockSpec((1, d2), lambda i: (0, 0)),
            pl.BlockSpec((d2, z_dim), lambda i: (0, 0)),
            pl.BlockSpec((1, z_dim), lambda i: (0, 0)),
            pl.BlockSpec((split_dim, k), lambda i: (0, 0)),
        ],
        out_specs=[
            pl.BlockSpec((bn, z_dim), lambda i: (i, 0)),
            pl.BlockSpec((split, bn), lambda i: (0, i)),
        ],
        out_shape=[
            jax.ShapeDtypeStruct((n, z_dim), jnp.float32),
            jax.ShapeDtypeStruct((split, n), jnp.int32),
        ],
    )(x, W1, b1r, W2, b2r, W3, b3r, ct)

    indices = idxs.T.astype(jnp.int64)
    return (z, indices)
